# baseline (speedup 1.0000x reference)
import functools

import numpy as np
import jax
import jax.numpy as jnp
from jax.experimental import pallas as pl
from jax.experimental.pallas import tpu as pltpu

_WIDTH = 128
_NUM_STEPS = 100
_B_BLK = 8
_TILES = _WIDTH // 8

_E = float(np.exp(-2.0))
_W_EDGE = _E / (1.0 + 2.0 * _E)
_W_CENTER = 1.0 / (1.0 + 2.0 * _E)


def _raster_kernel(tc_ref, strokes_ref, out_ref):
    width = _WIDTH
    nb = _B_BLK
    rowp = (
        8 * jax.lax.broadcasted_iota(jnp.int32, (8, width), 1)
        + jax.lax.broadcasted_iota(jnp.int32, (8, width), 0)
    ).astype(jnp.float32)
    col = jax.lax.broadcasted_iota(jnp.int32, (1, width), 1).astype(jnp.float32)

    we = jnp.float32(_W_EDGE)
    wc = jnp.float32(_W_CENTER)
    zrow = jnp.zeros((1, width), jnp.float32)
    zcol = jnp.zeros((width, 1), jnp.float32)

    for b in range(nb):
        a0x = strokes_ref[b, 0]
        a0y = strokes_ref[b, 1]
        f1x = strokes_ref[b, 2]
        f1y = strokes_ref[b, 3]
        a2x = strokes_ref[b, 4]
        a2y = strokes_ref[b, 5]
        b1x = a0x + (a2x - a0x) * f1x
        b1y = a0y + (a2y - a0y) * f1y
        scale = jnp.float32(width - 1)
        half = jnp.float32(0.5)
        p0x = a0x * scale + half
        p0y = a0y * scale + half
        p1x = b1x * scale + half
        p1y = b1y * scale + half
        p2x = a2x * scale + half
        p2y = a2y * scale + half
        r0 = jnp.float32(1.0) + strokes_ref[b, 6] * jnp.float32(width // 4)
        r2 = jnp.float32(1.0) + strokes_ref[b, 7] * jnp.float32(width // 4)
        c0 = strokes_ref[b, 8]
        c2 = strokes_ref[b, 9]

        def step(i, tiles):
            qs = tc_ref[0, i]
            qm = tc_ref[1, i]
            qe = tc_ref[2, i]
            tt = tc_ref[3, i]
            mt = tc_ref[4, i]
            x = p0x * qs + p1x * qm + p2x * qe
            y = p0y * qs + p1y * qm + p2y * qe
            rt = r0 * mt + r2 * tt
            ct = c0 * mt + c2 * tt
            dx = rowp - x
            thr = rt * rt - dx * dx
            dy2 = (col - y) * (col - y)
            return tuple(
                jnp.where(dy2 < thr[:, k : k + 1], ct, tiles[k])
                for k in range(_TILES)
            )

        tiles0 = tuple(jnp.zeros((8, width), jnp.float32) for _ in range(_TILES))
        tiles = jax.lax.fori_loop(0, _NUM_STEPS, step, tiles0)
        c = jnp.concatenate(tiles, axis=0)

        up = jnp.concatenate([c[1:, :], zrow], axis=0)
        down = jnp.concatenate([zrow, c[: width - 1, :]], axis=0)
        c = we * up + wc * c + we * down
        left = jnp.concatenate([c[:, 1:], zcol], axis=1)
        right = jnp.concatenate([zcol, c[:, : width - 1]], axis=1)
        out_ref[b] = jnp.float32(1.0) - (we * left + wc * c + we * right)


@jax.jit
def kernel(strokes):
    batch = strokes.shape[0]
    t = jnp.linspace(0.0, 1.0, _NUM_STEPS, dtype=jnp.float32)
    minus_t = 1.0 - t
    tcoef = jnp.stack([minus_t**2, 2.0 * minus_t * t, t**2, t, minus_t])

    grid = (batch // _B_BLK,)
    return pl.pallas_call(
        _raster_kernel,
        grid=grid,
        in_specs=[
            pl.BlockSpec((5, _NUM_STEPS), lambda i: (0, 0), memory_space=pltpu.SMEM),
            pl.BlockSpec((_B_BLK, 10), lambda i: (i, 0), memory_space=pltpu.SMEM),
        ],
        out_specs=pl.BlockSpec((_B_BLK, _WIDTH, _WIDTH), lambda i: (i, 0, 0)),
        out_shape=jax.ShapeDtypeStruct((batch, _WIDTH, _WIDTH), jnp.float32),
        compiler_params=pltpu.CompilerParams(
            dimension_semantics=("arbitrary",),
        ),
    )(tcoef, strokes)


# speedup vs baseline: 2.7171x; 2.7171x over previous
import functools

import numpy as np
import jax
import jax.numpy as jnp
from jax.experimental import pallas as pl
from jax.experimental.pallas import tpu as pltpu

_WIDTH = 128
_NUM_STEPS = 100
_B_BLK = 8
_TILES = _WIDTH // 8

_E = float(np.exp(-2.0))
_W_EDGE = _E / (1.0 + 2.0 * _E)
_W_CENTER = 1.0 / (1.0 + 2.0 * _E)


def _raster_kernel(tc_ref, strokes_ref, out_ref, canvas):
    width = _WIDTH
    nb = _B_BLK
    rowp = (
        8 * jax.lax.broadcasted_iota(jnp.int32, (8, width), 1)
        + jax.lax.broadcasted_iota(jnp.int32, (8, width), 0)
    ).astype(jnp.float32)
    col = jax.lax.broadcasted_iota(jnp.int32, (1, width), 1).astype(jnp.float32)

    canvas[...] = jnp.zeros((nb, width, width), jnp.float32)

    p0x, p0y, p1x, p1y, p2x, p2y = [], [], [], [], [], []
    r0, r2, c0, c2 = [], [], [], []
    for b in range(nb):
        a0x = strokes_ref[b, 0]
        a0y = strokes_ref[b, 1]
        f1x = strokes_ref[b, 2]
        f1y = strokes_ref[b, 3]
        a2x = strokes_ref[b, 4]
        a2y = strokes_ref[b, 5]
        b1x = a0x + (a2x - a0x) * f1x
        b1y = a0y + (a2y - a0y) * f1y
        scale = jnp.float32(width - 1)
        half = jnp.float32(0.5)
        p0x.append(a0x * scale + half)
        p0y.append(a0y * scale + half)
        p1x.append(b1x * scale + half)
        p1y.append(b1y * scale + half)
        p2x.append(a2x * scale + half)
        p2y.append(a2y * scale + half)
        r0.append(jnp.float32(1.0) + strokes_ref[b, 6] * jnp.float32(width // 4))
        r2.append(jnp.float32(1.0) + strokes_ref[b, 7] * jnp.float32(width // 4))
        c0.append(strokes_ref[b, 8])
        c2.append(strokes_ref[b, 9])

    def step(i, carry):
        qs = tc_ref[0, i]
        qm = tc_ref[1, i]
        qe = tc_ref[2, i]
        tt = tc_ref[3, i]
        mt = tc_ref[4, i]
        for b in range(nb):
            x = p0x[b] * qs + p1x[b] * qm + p2x[b] * qe
            y = p0y[b] * qs + p1y[b] * qm + p2y[b] * qe
            rt = r0[b] * mt + r2[b] * tt
            ct = c0[b] * mt + c2[b] * tt
            dx = rowp - x
            thr = rt * rt - dx * dx
            dy2 = (col - y) * (col - y)
            for k in range(_TILES):
                old = canvas[b, 8 * k : 8 * k + 8, :]
                canvas[b, 8 * k : 8 * k + 8, :] = jnp.where(
                    dy2 < thr[:, k : k + 1], ct, old
                )
        return carry

    jax.lax.fori_loop(0, _NUM_STEPS, step, 0)

    we = jnp.float32(_W_EDGE)
    wc = jnp.float32(_W_CENTER)
    zrow = jnp.zeros((1, width), jnp.float32)
    zcol = jnp.zeros((width, 1), jnp.float32)
    for b in range(nb):
        c = canvas[b]
        up = jnp.concatenate([c[1:, :], zrow], axis=0)
        down = jnp.concatenate([zrow, c[: width - 1, :]], axis=0)
        c = we * up + wc * c + we * down
        left = jnp.concatenate([c[:, 1:], zcol], axis=1)
        right = jnp.concatenate([zcol, c[:, : width - 1]], axis=1)
        out_ref[b] = jnp.float32(1.0) - (we * left + wc * c + we * right)


@jax.jit
def kernel(strokes):
    batch = strokes.shape[0]
    t = jnp.linspace(0.0, 1.0, _NUM_STEPS, dtype=jnp.float32)
    minus_t = 1.0 - t
    tcoef = jnp.stack([minus_t**2, 2.0 * minus_t * t, t**2, t, minus_t])

    grid = (batch // _B_BLK,)
    return pl.pallas_call(
        _raster_kernel,
        grid=grid,
        in_specs=[
            pl.BlockSpec((5, _NUM_STEPS), lambda i: (0, 0), memory_space=pltpu.SMEM),
            pl.BlockSpec((_B_BLK, 10), lambda i: (i, 0), memory_space=pltpu.SMEM),
        ],
        out_specs=pl.BlockSpec((_B_BLK, _WIDTH, _WIDTH), lambda i: (i, 0, 0)),
        out_shape=jax.ShapeDtypeStruct((batch, _WIDTH, _WIDTH), jnp.float32),
        scratch_shapes=[pltpu.VMEM((_B_BLK, _WIDTH, _WIDTH), jnp.float32)],
        compiler_params=pltpu.CompilerParams(
            dimension_semantics=("arbitrary",),
        ),
    )(tcoef, strokes)


# speedup vs baseline: 3.6467x; 1.3421x over previous
import functools

import numpy as np
import jax
import jax.numpy as jnp
from jax.experimental import pallas as pl
from jax.experimental.pallas import tpu as pltpu

_WIDTH = 128
_NUM_STEPS = 100
_B_BLK = 8
_TILES = _WIDTH // 8

_E = float(np.exp(-2.0))
_W_EDGE = _E / (1.0 + 2.0 * _E)
_W_CENTER = 1.0 / (1.0 + 2.0 * _E)


def _raster_kernel(tc_ref, strokes_ref, out_ref, canvas):
    width = _WIDTH
    nb = _B_BLK
    col = jax.lax.broadcasted_iota(jnp.int32, (1, width), 1).astype(jnp.float32)

    lane8 = jax.lax.broadcasted_iota(jnp.int32, (width, 8), 1)
    rowv = jax.lax.broadcasted_iota(jnp.int32, (width, 8), 0).astype(jnp.float32)
    vand = jnp.where(lane8 == 0, jnp.float32(1.0), jnp.float32(0.0))
    vand = jnp.where(lane8 == 1, rowv, vand)
    vand = jnp.where(lane8 == 2, rowv * rowv, vand)
    sub8 = jax.lax.broadcasted_iota(jnp.int32, (8, width), 0)
    e0 = jnp.where(sub8 == 0, jnp.float32(1.0), jnp.float32(0.0))
    e1 = jnp.where(sub8 == 1, jnp.float32(1.0), jnp.float32(0.0))
    e2neg = jnp.where(sub8 == 2, jnp.float32(-1.0), jnp.float32(0.0))

    canvas[...] = jnp.zeros((nb, width, width), jnp.float32)

    p0x, p0y, p1x, p1y, p2x, p2y = [], [], [], [], [], []
    r0, r2, c0, c2 = [], [], [], []
    for b in range(nb):
        a0x = strokes_ref[b, 0]
        a0y = strokes_ref[b, 1]
        f1x = strokes_ref[b, 2]
        f1y = strokes_ref[b, 3]
        a2x = strokes_ref[b, 4]
        a2y = strokes_ref[b, 5]
        b1x = a0x + (a2x - a0x) * f1x
        b1y = a0y + (a2y - a0y) * f1y
        scale = jnp.float32(width - 1)
        half = jnp.float32(0.5)
        p0x.append(a0x * scale + half)
        p0y.append(a0y * scale + half)
        p1x.append(b1x * scale + half)
        p1y.append(b1y * scale + half)
        p2x.append(a2x * scale + half)
        p2y.append(a2y * scale + half)
        r0.append(jnp.float32(1.0) + strokes_ref[b, 6] * jnp.float32(width // 4))
        r2.append(jnp.float32(1.0) + strokes_ref[b, 7] * jnp.float32(width // 4))
        c0.append(strokes_ref[b, 8])
        c2.append(strokes_ref[b, 9])

    def step(i, carry):
        qs = tc_ref[0, i]
        qm = tc_ref[1, i]
        qe = tc_ref[2, i]
        tt = tc_ref[3, i]
        mt = tc_ref[4, i]
        for b in range(nb):
            x = p0x[b] * qs + p1x[b] * qm + p2x[b] * qe
            y = p0y[b] * qs + p1y[b] * qm + p2y[b] * qe
            rt = r0[b] * mt + r2[b] * tt
            ct = c0[b] * mt + c2[b] * tt
            coef = (rt * rt - x * x) * e0 + (x + x) * e1 + e2neg
            thr = jnp.dot(vand, coef, preferred_element_type=jnp.float32)
            dy2 = (col - y) * (col - y)
            mask = dy2 < thr
            canvas[b] = jnp.where(mask, ct, canvas[b])
        return carry

    jax.lax.fori_loop(0, _NUM_STEPS, step, 0)

    we = jnp.float32(_W_EDGE)
    wc = jnp.float32(_W_CENTER)
    zrow = jnp.zeros((1, width), jnp.float32)
    zcol = jnp.zeros((width, 1), jnp.float32)
    for b in range(nb):
        c = canvas[b]
        up = jnp.concatenate([c[1:, :], zrow], axis=0)
        down = jnp.concatenate([zrow, c[: width - 1, :]], axis=0)
        c = we * up + wc * c + we * down
        left = jnp.concatenate([c[:, 1:], zcol], axis=1)
        right = jnp.concatenate([zcol, c[:, : width - 1]], axis=1)
        out_ref[b] = jnp.float32(1.0) - (we * left + wc * c + we * right)


@jax.jit
def kernel(strokes):
    batch = strokes.shape[0]
    t = jnp.linspace(0.0, 1.0, _NUM_STEPS, dtype=jnp.float32)
    minus_t = 1.0 - t
    tcoef = jnp.stack([minus_t**2, 2.0 * minus_t * t, t**2, t, minus_t])

    grid = (batch // _B_BLK,)
    return pl.pallas_call(
        _raster_kernel,
        grid=grid,
        in_specs=[
            pl.BlockSpec((5, _NUM_STEPS), lambda i: (0, 0), memory_space=pltpu.SMEM),
            pl.BlockSpec((_B_BLK, 10), lambda i: (i, 0), memory_space=pltpu.SMEM),
        ],
        out_specs=pl.BlockSpec((_B_BLK, _WIDTH, _WIDTH), lambda i: (i, 0, 0)),
        out_shape=jax.ShapeDtypeStruct((batch, _WIDTH, _WIDTH), jnp.float32),
        scratch_shapes=[pltpu.VMEM((_B_BLK, _WIDTH, _WIDTH), jnp.float32)],
        compiler_params=pltpu.CompilerParams(
            dimension_semantics=("arbitrary",),
        ),
    )(tcoef, strokes)


# speedup vs baseline: 7.3770x; 2.0229x over previous
import functools

import numpy as np
import jax
import jax.numpy as jnp
from jax.experimental import pallas as pl
from jax.experimental.pallas import tpu as pltpu

_WIDTH = 128
_NUM_STEPS = 100
_B_BLK = 8
_TILES = _WIDTH // 8

_E = float(np.exp(-2.0))
_W_EDGE = _E / (1.0 + 2.0 * _E)
_W_CENTER = 1.0 / (1.0 + 2.0 * _E)


def _raster_kernel(tc_ref, strokes_ref, out_ref, canvas):
    width = _WIDTH
    nb = _B_BLK
    col = jax.lax.broadcasted_iota(jnp.int32, (1, width), 1).astype(jnp.float32)

    row = jax.lax.broadcasted_iota(jnp.int32, (width, 1), 0).astype(jnp.float32)

    canvas[...] = jnp.zeros((nb, width, width), jnp.float32)

    p0x, p0y, p1x, p1y, p2x, p2y = [], [], [], [], [], []
    r0, r2, c0, c2 = [], [], [], []
    for b in range(nb):
        a0x = strokes_ref[b, 0]
        a0y = strokes_ref[b, 1]
        f1x = strokes_ref[b, 2]
        f1y = strokes_ref[b, 3]
        a2x = strokes_ref[b, 4]
        a2y = strokes_ref[b, 5]
        b1x = a0x + (a2x - a0x) * f1x
        b1y = a0y + (a2y - a0y) * f1y
        scale = jnp.float32(width - 1)
        half = jnp.float32(0.5)
        p0x.append(a0x * scale + half)
        p0y.append(a0y * scale + half)
        p1x.append(b1x * scale + half)
        p1y.append(b1y * scale + half)
        p2x.append(a2x * scale + half)
        p2y.append(a2y * scale + half)
        r0.append(jnp.float32(1.0) + strokes_ref[b, 6] * jnp.float32(width // 4))
        r2.append(jnp.float32(1.0) + strokes_ref[b, 7] * jnp.float32(width // 4))
        c0.append(strokes_ref[b, 8])
        c2.append(strokes_ref[b, 9])

    def step(i, carry):
        qs = tc_ref[0, i]
        qm = tc_ref[1, i]
        qe = tc_ref[2, i]
        tt = tc_ref[3, i]
        mt = tc_ref[4, i]
        for b in range(nb):
            x = p0x[b] * qs + p1x[b] * qm + p2x[b] * qe
            y = p0y[b] * qs + p1y[b] * qm + p2y[b] * qe
            rt = r0[b] * mt + r2[b] * tt
            ct = c0[b] * mt + c2[b] * tt
            dx2 = (row - x) * (row - x)
            dy2 = (col - y) * (col - y)
            mask = (dx2 + dy2) < rt * rt
            canvas[b] = jnp.where(mask, ct, canvas[b])
        return carry

    jax.lax.fori_loop(0, _NUM_STEPS, step, 0)

    we = jnp.float32(_W_EDGE)
    wc = jnp.float32(_W_CENTER)
    zrow = jnp.zeros((1, width), jnp.float32)
    zcol = jnp.zeros((width, 1), jnp.float32)
    for b in range(nb):
        c = canvas[b]
        up = jnp.concatenate([c[1:, :], zrow], axis=0)
        down = jnp.concatenate([zrow, c[: width - 1, :]], axis=0)
        c = we * up + wc * c + we * down
        left = jnp.concatenate([c[:, 1:], zcol], axis=1)
        right = jnp.concatenate([zcol, c[:, : width - 1]], axis=1)
        out_ref[b] = jnp.float32(1.0) - (we * left + wc * c + we * right)


@jax.jit
def kernel(strokes):
    batch = strokes.shape[0]
    t = jnp.linspace(0.0, 1.0, _NUM_STEPS, dtype=jnp.float32)
    minus_t = 1.0 - t
    tcoef = jnp.stack([minus_t**2, 2.0 * minus_t * t, t**2, t, minus_t])

    grid = (batch // _B_BLK,)
    return pl.pallas_call(
        _raster_kernel,
        grid=grid,
        in_specs=[
            pl.BlockSpec((5, _NUM_STEPS), lambda i: (0, 0), memory_space=pltpu.SMEM),
            pl.BlockSpec((_B_BLK, 10), lambda i: (i, 0), memory_space=pltpu.SMEM),
        ],
        out_specs=pl.BlockSpec((_B_BLK, _WIDTH, _WIDTH), lambda i: (i, 0, 0)),
        out_shape=jax.ShapeDtypeStruct((batch, _WIDTH, _WIDTH), jnp.float32),
        scratch_shapes=[pltpu.VMEM((_B_BLK, _WIDTH, _WIDTH), jnp.float32)],
        compiler_params=pltpu.CompilerParams(
            dimension_semantics=("arbitrary",),
        ),
    )(tcoef, strokes)


# speedup vs baseline: 7.9449x; 1.0770x over previous
import functools

import numpy as np
import jax
import jax.numpy as jnp
from jax.experimental import pallas as pl
from jax.experimental.pallas import tpu as pltpu

_WIDTH = 128
_NUM_STEPS = 100
_B_BLK = 16
_TILES = _WIDTH // 8

_E = float(np.exp(-2.0))
_W_EDGE = _E / (1.0 + 2.0 * _E)
_W_CENTER = 1.0 / (1.0 + 2.0 * _E)


def _raster_kernel(tc_ref, strokes_ref, out_ref, canvas):
    width = _WIDTH
    nb = _B_BLK
    col = jax.lax.broadcasted_iota(jnp.int32, (1, width), 1).astype(jnp.float32)

    row = jax.lax.broadcasted_iota(jnp.int32, (width, 1), 0).astype(jnp.float32)

    canvas[...] = jnp.zeros((nb, width, width), jnp.float32)

    p0x, p0y, p1x, p1y, p2x, p2y = [], [], [], [], [], []
    r0, r2, c0, c2 = [], [], [], []
    for b in range(nb):
        a0x = strokes_ref[b, 0]
        a0y = strokes_ref[b, 1]
        f1x = strokes_ref[b, 2]
        f1y = strokes_ref[b, 3]
        a2x = strokes_ref[b, 4]
        a2y = strokes_ref[b, 5]
        b1x = a0x + (a2x - a0x) * f1x
        b1y = a0y + (a2y - a0y) * f1y
        scale = jnp.float32(width - 1)
        half = jnp.float32(0.5)
        p0x.append(a0x * scale + half)
        p0y.append(a0y * scale + half)
        p1x.append(b1x * scale + half)
        p1y.append(b1y * scale + half)
        p2x.append(a2x * scale + half)
        p2y.append(a2y * scale + half)
        r0.append(jnp.float32(1.0) + strokes_ref[b, 6] * jnp.float32(width // 4))
        r2.append(jnp.float32(1.0) + strokes_ref[b, 7] * jnp.float32(width // 4))
        c0.append(strokes_ref[b, 8])
        c2.append(strokes_ref[b, 9])

    def step(i, carry):
        qs = tc_ref[0, i]
        qm = tc_ref[1, i]
        qe = tc_ref[2, i]
        tt = tc_ref[3, i]
        mt = tc_ref[4, i]
        for b in range(nb):
            x = p0x[b] * qs + p1x[b] * qm + p2x[b] * qe
            y = p0y[b] * qs + p1y[b] * qm + p2y[b] * qe
            rt = r0[b] * mt + r2[b] * tt
            ct = c0[b] * mt + c2[b] * tt
            dx2 = (row - x) * (row - x)
            dy2 = (col - y) * (col - y)
            mask = (dx2 + dy2) < rt * rt
            canvas[b] = jnp.where(mask, ct, canvas[b])
        return carry

    jax.lax.fori_loop(0, _NUM_STEPS, step, 0)

    we = jnp.float32(_W_EDGE)
    wc = jnp.float32(_W_CENTER)
    zrow = jnp.zeros((1, width), jnp.float32)
    zcol = jnp.zeros((width, 1), jnp.float32)
    for b in range(nb):
        c = canvas[b]
        up = jnp.concatenate([c[1:, :], zrow], axis=0)
        down = jnp.concatenate([zrow, c[: width - 1, :]], axis=0)
        c = we * up + wc * c + we * down
        left = jnp.concatenate([c[:, 1:], zcol], axis=1)
        right = jnp.concatenate([zcol, c[:, : width - 1]], axis=1)
        out_ref[b] = jnp.float32(1.0) - (we * left + wc * c + we * right)


@jax.jit
def kernel(strokes):
    batch = strokes.shape[0]
    t = jnp.linspace(0.0, 1.0, _NUM_STEPS, dtype=jnp.float32)
    minus_t = 1.0 - t
    tcoef = jnp.stack([minus_t**2, 2.0 * minus_t * t, t**2, t, minus_t])

    grid = (batch // _B_BLK,)
    return pl.pallas_call(
        _raster_kernel,
        grid=grid,
        in_specs=[
            pl.BlockSpec((5, _NUM_STEPS), lambda i: (0, 0), memory_space=pltpu.SMEM),
            pl.BlockSpec((_B_BLK, 10), lambda i: (i, 0), memory_space=pltpu.SMEM),
        ],
        out_specs=pl.BlockSpec((_B_BLK, _WIDTH, _WIDTH), lambda i: (i, 0, 0)),
        out_shape=jax.ShapeDtypeStruct((batch, _WIDTH, _WIDTH), jnp.float32),
        scratch_shapes=[pltpu.VMEM((_B_BLK, _WIDTH, _WIDTH), jnp.float32)],
        compiler_params=pltpu.CompilerParams(
            dimension_semantics=("arbitrary",),
        ),
    )(tcoef, strokes)


# speedup vs baseline: 8.2645x; 1.0402x over previous
import functools

import numpy as np
import jax
import jax.numpy as jnp
from jax.experimental import pallas as pl
from jax.experimental.pallas import tpu as pltpu

_WIDTH = 128
_NUM_STEPS = 100
_B_BLK = 32
_TILES = _WIDTH // 8

_E = float(np.exp(-2.0))
_W_EDGE = _E / (1.0 + 2.0 * _E)
_W_CENTER = 1.0 / (1.0 + 2.0 * _E)


def _raster_kernel(tc_ref, strokes_ref, out_ref, canvas):
    width = _WIDTH
    nb = _B_BLK
    col = jax.lax.broadcasted_iota(jnp.int32, (1, width), 1).astype(jnp.float32)

    row = jax.lax.broadcasted_iota(jnp.int32, (width, 1), 0).astype(jnp.float32)

    canvas[...] = jnp.zeros((nb, width, width), jnp.float32)

    p0x, p0y, p1x, p1y, p2x, p2y = [], [], [], [], [], []
    r0, r2, c0, c2 = [], [], [], []
    for b in range(nb):
        a0x = strokes_ref[b, 0]
        a0y = strokes_ref[b, 1]
        f1x = strokes_ref[b, 2]
        f1y = strokes_ref[b, 3]
        a2x = strokes_ref[b, 4]
        a2y = strokes_ref[b, 5]
        b1x = a0x + (a2x - a0x) * f1x
        b1y = a0y + (a2y - a0y) * f1y
        scale = jnp.float32(width - 1)
        half = jnp.float32(0.5)
        p0x.append(a0x * scale + half)
        p0y.append(a0y * scale + half)
        p1x.append(b1x * scale + half)
        p1y.append(b1y * scale + half)
        p2x.append(a2x * scale + half)
        p2y.append(a2y * scale + half)
        r0.append(jnp.float32(1.0) + strokes_ref[b, 6] * jnp.float32(width // 4))
        r2.append(jnp.float32(1.0) + strokes_ref[b, 7] * jnp.float32(width // 4))
        c0.append(strokes_ref[b, 8])
        c2.append(strokes_ref[b, 9])

    def step(i, carry):
        qs = tc_ref[0, i]
        qm = tc_ref[1, i]
        qe = tc_ref[2, i]
        tt = tc_ref[3, i]
        mt = tc_ref[4, i]
        for b in range(nb):
            x = p0x[b] * qs + p1x[b] * qm + p2x[b] * qe
            y = p0y[b] * qs + p1y[b] * qm + p2y[b] * qe
            rt = r0[b] * mt + r2[b] * tt
            ct = c0[b] * mt + c2[b] * tt
            dx2 = (row - x) * (row - x)
            dy2 = (col - y) * (col - y)
            mask = (dx2 + dy2) < rt * rt
            canvas[b] = jnp.where(mask, ct, canvas[b])
        return carry

    jax.lax.fori_loop(0, _NUM_STEPS, step, 0)

    we = jnp.float32(_W_EDGE)
    wc = jnp.float32(_W_CENTER)
    zrow = jnp.zeros((1, width), jnp.float32)
    zcol = jnp.zeros((width, 1), jnp.float32)
    for b in range(nb):
        c = canvas[b]
        up = jnp.concatenate([c[1:, :], zrow], axis=0)
        down = jnp.concatenate([zrow, c[: width - 1, :]], axis=0)
        c = we * up + wc * c + we * down
        left = jnp.concatenate([c[:, 1:], zcol], axis=1)
        right = jnp.concatenate([zcol, c[:, : width - 1]], axis=1)
        out_ref[b] = jnp.float32(1.0) - (we * left + wc * c + we * right)


@jax.jit
def kernel(strokes):
    batch = strokes.shape[0]
    t = jnp.linspace(0.0, 1.0, _NUM_STEPS, dtype=jnp.float32)
    minus_t = 1.0 - t
    tcoef = jnp.stack([minus_t**2, 2.0 * minus_t * t, t**2, t, minus_t])

    grid = (batch // _B_BLK,)
    return pl.pallas_call(
        _raster_kernel,
        grid=grid,
        in_specs=[
            pl.BlockSpec((5, _NUM_STEPS), lambda i: (0, 0), memory_space=pltpu.SMEM),
            pl.BlockSpec((_B_BLK, 10), lambda i: (i, 0), memory_space=pltpu.SMEM),
        ],
        out_specs=pl.BlockSpec((_B_BLK, _WIDTH, _WIDTH), lambda i: (i, 0, 0)),
        out_shape=jax.ShapeDtypeStruct((batch, _WIDTH, _WIDTH), jnp.float32),
        scratch_shapes=[pltpu.VMEM((_B_BLK, _WIDTH, _WIDTH), jnp.float32)],
        compiler_params=pltpu.CompilerParams(
            dimension_semantics=("arbitrary",),
        ),
    )(tcoef, strokes)


# speedup vs baseline: 8.4432x; 1.0216x over previous
import functools

import numpy as np
import jax
import jax.numpy as jnp
from jax.experimental import pallas as pl
from jax.experimental.pallas import tpu as pltpu

_WIDTH = 128
_NUM_STEPS = 100
_B_BLK = 64
_TILES = _WIDTH // 8

_E = float(np.exp(-2.0))
_W_EDGE = _E / (1.0 + 2.0 * _E)
_W_CENTER = 1.0 / (1.0 + 2.0 * _E)


def _raster_kernel(tc_ref, strokes_ref, out_ref, canvas):
    width = _WIDTH
    nb = _B_BLK
    col = jax.lax.broadcasted_iota(jnp.int32, (1, width), 1).astype(jnp.float32)

    row = jax.lax.broadcasted_iota(jnp.int32, (width, 1), 0).astype(jnp.float32)

    canvas[...] = jnp.zeros((nb, width, width), jnp.float32)

    p0x, p0y, p1x, p1y, p2x, p2y = [], [], [], [], [], []
    r0, r2, c0, c2 = [], [], [], []
    for b in range(nb):
        a0x = strokes_ref[b, 0]
        a0y = strokes_ref[b, 1]
        f1x = strokes_ref[b, 2]
        f1y = strokes_ref[b, 3]
        a2x = strokes_ref[b, 4]
        a2y = strokes_ref[b, 5]
        b1x = a0x + (a2x - a0x) * f1x
        b1y = a0y + (a2y - a0y) * f1y
        scale = jnp.float32(width - 1)
        half = jnp.float32(0.5)
        p0x.append(a0x * scale + half)
        p0y.append(a0y * scale + half)
        p1x.append(b1x * scale + half)
        p1y.append(b1y * scale + half)
        p2x.append(a2x * scale + half)
        p2y.append(a2y * scale + half)
        r0.append(jnp.float32(1.0) + strokes_ref[b, 6] * jnp.float32(width // 4))
        r2.append(jnp.float32(1.0) + strokes_ref[b, 7] * jnp.float32(width // 4))
        c0.append(strokes_ref[b, 8])
        c2.append(strokes_ref[b, 9])

    def step(i, carry):
        qs = tc_ref[0, i]
        qm = tc_ref[1, i]
        qe = tc_ref[2, i]
        tt = tc_ref[3, i]
        mt = tc_ref[4, i]
        for b in range(nb):
            x = p0x[b] * qs + p1x[b] * qm + p2x[b] * qe
            y = p0y[b] * qs + p1y[b] * qm + p2y[b] * qe
            rt = r0[b] * mt + r2[b] * tt
            ct = c0[b] * mt + c2[b] * tt
            dx2 = (row - x) * (row - x)
            dy2 = (col - y) * (col - y)
            mask = (dx2 + dy2) < rt * rt
            canvas[b] = jnp.where(mask, ct, canvas[b])
        return carry

    jax.lax.fori_loop(0, _NUM_STEPS, step, 0)

    we = jnp.float32(_W_EDGE)
    wc = jnp.float32(_W_CENTER)
    zrow = jnp.zeros((1, width), jnp.float32)
    zcol = jnp.zeros((width, 1), jnp.float32)
    for b in range(nb):
        c = canvas[b]
        up = jnp.concatenate([c[1:, :], zrow], axis=0)
        down = jnp.concatenate([zrow, c[: width - 1, :]], axis=0)
        c = we * up + wc * c + we * down
        left = jnp.concatenate([c[:, 1:], zcol], axis=1)
        right = jnp.concatenate([zcol, c[:, : width - 1]], axis=1)
        out_ref[b] = jnp.float32(1.0) - (we * left + wc * c + we * right)


@jax.jit
def kernel(strokes):
    batch = strokes.shape[0]
    t = jnp.linspace(0.0, 1.0, _NUM_STEPS, dtype=jnp.float32)
    minus_t = 1.0 - t
    tcoef = jnp.stack([minus_t**2, 2.0 * minus_t * t, t**2, t, minus_t])

    grid = (batch // _B_BLK,)
    return pl.pallas_call(
        _raster_kernel,
        grid=grid,
        in_specs=[
            pl.BlockSpec((5, _NUM_STEPS), lambda i: (0, 0), memory_space=pltpu.SMEM),
            pl.BlockSpec((_B_BLK, 10), lambda i: (i, 0), memory_space=pltpu.SMEM),
        ],
        out_specs=pl.BlockSpec((_B_BLK, _WIDTH, _WIDTH), lambda i: (i, 0, 0)),
        out_shape=jax.ShapeDtypeStruct((batch, _WIDTH, _WIDTH), jnp.float32),
        scratch_shapes=[pltpu.VMEM((_B_BLK, _WIDTH, _WIDTH), jnp.float32)],
        compiler_params=pltpu.CompilerParams(
            dimension_semantics=("arbitrary",),
        ),
    )(tcoef, strokes)


# speedup vs baseline: 8.5114x; 1.0081x over previous
import functools

import numpy as np
import jax
import jax.numpy as jnp
from jax.experimental import pallas as pl
from jax.experimental.pallas import tpu as pltpu

_WIDTH = 128
_NUM_STEPS = 100
_B_BLK = 128
_TILES = _WIDTH // 8

_E = float(np.exp(-2.0))
_W_EDGE = _E / (1.0 + 2.0 * _E)
_W_CENTER = 1.0 / (1.0 + 2.0 * _E)


def _raster_kernel(tc_ref, strokes_ref, out_ref, canvas):
    width = _WIDTH
    nb = _B_BLK
    col = jax.lax.broadcasted_iota(jnp.int32, (1, width), 1).astype(jnp.float32)

    row = jax.lax.broadcasted_iota(jnp.int32, (width, 1), 0).astype(jnp.float32)

    canvas[...] = jnp.zeros((nb, width, width), jnp.float32)

    p0x, p0y, p1x, p1y, p2x, p2y = [], [], [], [], [], []
    r0, r2, c0, c2 = [], [], [], []
    for b in range(nb):
        a0x = strokes_ref[b, 0]
        a0y = strokes_ref[b, 1]
        f1x = strokes_ref[b, 2]
        f1y = strokes_ref[b, 3]
        a2x = strokes_ref[b, 4]
        a2y = strokes_ref[b, 5]
        b1x = a0x + (a2x - a0x) * f1x
        b1y = a0y + (a2y - a0y) * f1y
        scale = jnp.float32(width - 1)
        half = jnp.float32(0.5)
        p0x.append(a0x * scale + half)
        p0y.append(a0y * scale + half)
        p1x.append(b1x * scale + half)
        p1y.append(b1y * scale + half)
        p2x.append(a2x * scale + half)
        p2y.append(a2y * scale + half)
        r0.append(jnp.float32(1.0) + strokes_ref[b, 6] * jnp.float32(width // 4))
        r2.append(jnp.float32(1.0) + strokes_ref[b, 7] * jnp.float32(width // 4))
        c0.append(strokes_ref[b, 8])
        c2.append(strokes_ref[b, 9])

    def step(i, carry):
        qs = tc_ref[0, i]
        qm = tc_ref[1, i]
        qe = tc_ref[2, i]
        tt = tc_ref[3, i]
        mt = tc_ref[4, i]
        for b in range(nb):
            x = p0x[b] * qs + p1x[b] * qm + p2x[b] * qe
            y = p0y[b] * qs + p1y[b] * qm + p2y[b] * qe
            rt = r0[b] * mt + r2[b] * tt
            ct = c0[b] * mt + c2[b] * tt
            dx2 = (row - x) * (row - x)
            dy2 = (col - y) * (col - y)
            mask = (dx2 + dy2) < rt * rt
            canvas[b] = jnp.where(mask, ct, canvas[b])
        return carry

    jax.lax.fori_loop(0, _NUM_STEPS, step, 0)

    we = jnp.float32(_W_EDGE)
    wc = jnp.float32(_W_CENTER)
    zrow = jnp.zeros((1, width), jnp.float32)
    zcol = jnp.zeros((width, 1), jnp.float32)
    for b in range(nb):
        c = canvas[b]
        up = jnp.concatenate([c[1:, :], zrow], axis=0)
        down = jnp.concatenate([zrow, c[: width - 1, :]], axis=0)
        c = we * up + wc * c + we * down
        left = jnp.concatenate([c[:, 1:], zcol], axis=1)
        right = jnp.concatenate([zcol, c[:, : width - 1]], axis=1)
        out_ref[b] = jnp.float32(1.0) - (we * left + wc * c + we * right)


@jax.jit
def kernel(strokes):
    batch = strokes.shape[0]
    t = jnp.linspace(0.0, 1.0, _NUM_STEPS, dtype=jnp.float32)
    minus_t = 1.0 - t
    tcoef = jnp.stack([minus_t**2, 2.0 * minus_t * t, t**2, t, minus_t])

    grid = (batch // _B_BLK,)
    return pl.pallas_call(
        _raster_kernel,
        grid=grid,
        in_specs=[
            pl.BlockSpec((5, _NUM_STEPS), lambda i: (0, 0), memory_space=pltpu.SMEM),
            pl.BlockSpec((_B_BLK, 10), lambda i: (i, 0), memory_space=pltpu.SMEM),
        ],
        out_specs=pl.BlockSpec((_B_BLK, _WIDTH, _WIDTH), lambda i: (i, 0, 0)),
        out_shape=jax.ShapeDtypeStruct((batch, _WIDTH, _WIDTH), jnp.float32),
        scratch_shapes=[pltpu.VMEM((_B_BLK, _WIDTH, _WIDTH), jnp.float32)],
        compiler_params=pltpu.CompilerParams(
            dimension_semantics=("arbitrary",),
        ),
    )(tcoef, strokes)


# speedup vs baseline: 8.5418x; 1.0036x over previous
import numpy as np
import jax
import jax.numpy as jnp
from jax.experimental import pallas as pl
from jax.experimental.pallas import tpu as pltpu

_WIDTH = 128
_NUM_STEPS = 100
_B_BLK = 128
_TILES = _WIDTH // 8

_E = float(np.exp(-2.0))
_W_EDGE = _E / (1.0 + 2.0 * _E)
_W_CENTER = 1.0 / (1.0 + 2.0 * _E)


def _raster_kernel(tc_ref, strokes_ref, out_ref, canvas):
    width = _WIDTH
    nb = _B_BLK
    col = jax.lax.broadcasted_iota(jnp.int32, (1, width), 1).astype(jnp.float32)

    row = jax.lax.broadcasted_iota(jnp.int32, (width, 1), 0).astype(jnp.float32)

    canvas[...] = jnp.zeros((nb, width, width), jnp.float32)

    p0x, p0y, p1x, p1y, p2x, p2y = [], [], [], [], [], []
    r0, r2, c0, c2 = [], [], [], []
    for b in range(nb):
        a0x = strokes_ref[b, 0]
        a0y = strokes_ref[b, 1]
        f1x = strokes_ref[b, 2]
        f1y = strokes_ref[b, 3]
        a2x = strokes_ref[b, 4]
        a2y = strokes_ref[b, 5]
        b1x = a0x + (a2x - a0x) * f1x
        b1y = a0y + (a2y - a0y) * f1y
        scale = jnp.float32(width - 1)
        half = jnp.float32(0.5)
        p0x.append(a0x * scale + half)
        p0y.append(a0y * scale + half)
        p1x.append(b1x * scale + half)
        p1y.append(b1y * scale + half)
        p2x.append(a2x * scale + half)
        p2y.append(a2y * scale + half)
        r0.append(jnp.float32(1.0) + strokes_ref[b, 6] * jnp.float32(width // 4))
        r2.append(jnp.float32(1.0) + strokes_ref[b, 7] * jnp.float32(width // 4))
        c0.append(strokes_ref[b, 8])
        c2.append(strokes_ref[b, 9])

    def step(i, carry):
        qs = tc_ref[0, i]
        qm = tc_ref[1, i]
        qe = tc_ref[2, i]
        tt = tc_ref[3, i]
        mt = tc_ref[4, i]
        for b in range(nb):
            x = p0x[b] * qs + p1x[b] * qm + p2x[b] * qe
            y = p0y[b] * qs + p1y[b] * qm + p2y[b] * qe
            rt = r0[b] * mt + r2[b] * tt
            ct = c0[b] * mt + c2[b] * tt
            dx2 = (row - x) * (row - x)
            dy2 = (col - y) * (col - y)
            mask = (dx2 + dy2) < rt * rt
            canvas[b] = jnp.where(mask, ct, canvas[b])
        return carry

    jax.lax.fori_loop(0, _NUM_STEPS, step, 0)

    we = jnp.float32(_W_EDGE)
    wc = jnp.float32(_W_CENTER)
    zrow = jnp.zeros((1, width), jnp.float32)
    zcol = jnp.zeros((width, 1), jnp.float32)
    for b in range(nb):
        c = canvas[b]
        up = jnp.concatenate([c[1:, :], zrow], axis=0)
        down = jnp.concatenate([zrow, c[: width - 1, :]], axis=0)
        c = we * up + wc * c + we * down
        left = jnp.concatenate([c[:, 1:], zcol], axis=1)
        right = jnp.concatenate([zcol, c[:, : width - 1]], axis=1)
        out_ref[b] = jnp.float32(1.0) - (we * left + wc * c + we * right)


@jax.jit
def kernel(strokes):
    batch = strokes.shape[0]
    t = jnp.linspace(0.0, 1.0, _NUM_STEPS, dtype=jnp.float32)
    minus_t = 1.0 - t
    tcoef = jnp.stack([minus_t**2, 2.0 * minus_t * t, t**2, t, minus_t])

    grid = (batch // _B_BLK,)
    return pl.pallas_call(
        _raster_kernel,
        grid=grid,
        in_specs=[
            pl.BlockSpec((5, _NUM_STEPS), lambda i: (0, 0), memory_space=pltpu.SMEM),
            pl.BlockSpec((_B_BLK, 10), lambda i: (i, 0), memory_space=pltpu.SMEM),
        ],
        out_specs=pl.BlockSpec((_B_BLK, _WIDTH, _WIDTH), lambda i: (i, 0, 0)),
        out_shape=jax.ShapeDtypeStruct((batch, _WIDTH, _WIDTH), jnp.float32),
        scratch_shapes=[pltpu.VMEM((_B_BLK, _WIDTH, _WIDTH), jnp.float32)],
        compiler_params=pltpu.CompilerParams(
            dimension_semantics=("arbitrary",),
        ),
    )(tcoef, strokes)
